# baseline (speedup 1.0000x reference)
import functools

import jax
import jax.numpy as jnp
from jax.experimental import pallas as pl
from jax.experimental.pallas import tpu as pltpu
from jax.experimental.pallas import tpu_sc as plsc

VOCAB = 1000
EMB = 128
FIELDS = 26

NB = 16


def kernel(indices, tables):
    batch = indices.shape[0]
    flat_tables = tables.reshape(FIELDS * VOCAB, EMB)
    offs = (jnp.arange(FIELDS, dtype=indices.dtype) * VOCAB)[None, :]
    idx3 = (indices + offs).reshape(batch // NB, NB, FIELDS).transpose(0, 2, 1)

    mesh = plsc.VectorSubcoreMesh(core_axis_name="core", subcore_axis_name="subcore")

    @functools.partial(
        pl.kernel,
        out_type=jax.ShapeDtypeStruct((batch, FIELDS * EMB), tables.dtype),
        mesh=mesh,
    )
    def gather_kernel(x_hbm, i_hbm, o_hbm):
        def body(i_vmem, o_vmem):
            @pl.loop(0, FIELDS)
            def _(f):
                pltpu.sync_copy(
                    x_hbm.at[i_vmem.at[0, f]],
                    o_vmem.at[:, pl.ds(f * EMB, EMB)],
                )

        pltpu.emit_pipeline(
            body,
            grid=(batch // NB,),
            in_specs=[pl.BlockSpec((1, FIELDS, NB), index_map=lambda i: (i, 0, 0))],
            out_specs=[pl.BlockSpec((NB, FIELDS * EMB), index_map=lambda i: (i, 0))],
            core_axis_name=("core", "subcore"),
            dimension_semantics=(pltpu.PARALLEL,),
        )(i_hbm, o_hbm)

    return gather_kernel(flat_tables, idx3)


# speedup vs baseline: 2.9465x; 2.9465x over previous
import functools

import jax
import jax.numpy as jnp
from jax.experimental import pallas as pl
from jax.experimental.pallas import tpu as pltpu
from jax.experimental.pallas import tpu_sc as plsc

VOCAB = 1000
EMB = 128
FIELDS = 26
SUB = 8

G = 2


def kernel(indices, tables):
    batch = indices.shape[0]
    ngrp = batch // SUB
    win = G * SUB * FIELDS
    flat_tables = tables.reshape(FIELDS * VOCAB, EMB)
    offs = (jnp.arange(FIELDS, dtype=indices.dtype) * VOCAB)[None, :]
    pidx = (indices + offs).reshape(ngrp, SUB, FIELDS).transpose(0, 2, 1)
    pidx = pidx.reshape(ngrp // G, 1, win)

    mesh = plsc.VectorSubcoreMesh(core_axis_name="core", subcore_axis_name="subcore")

    @functools.partial(
        pl.kernel,
        out_type=jax.ShapeDtypeStruct((ngrp, FIELDS, SUB, EMB), tables.dtype),
        mesh=mesh,
    )
    def gather_kernel(x_hbm, i_hbm, o_hbm):
        def body(i_vmem, o_vmem):
            pltpu.sync_copy(x_hbm.at[i_vmem.at[0, 0]], o_vmem.reshape(win, EMB))

        pltpu.emit_pipeline(
            body,
            grid=(ngrp // G,),
            in_specs=[pl.BlockSpec((1, 1, win), index_map=lambda i: (i, 0, 0))],
            out_specs=[
                pl.BlockSpec((G, FIELDS, SUB, EMB), index_map=lambda i: (i, 0, 0, 0))
            ],
            core_axis_name=("core", "subcore"),
            dimension_semantics=(pltpu.PARALLEL,),
        )(i_hbm, o_hbm)

    out4 = gather_kernel(flat_tables, pidx)
    return out4.transpose(0, 2, 1, 3).reshape(batch, FIELDS * EMB)


# speedup vs baseline: 2.9773x; 1.0105x over previous
import functools

import jax
import jax.numpy as jnp
from jax.experimental import pallas as pl
from jax.experimental.pallas import tpu as pltpu
from jax.experimental.pallas import tpu_sc as plsc

VOCAB = 1000
EMB = 128
FIELDS = 26
SUB = 8

G = 2


def kernel(indices, tables):
    batch = indices.shape[0]
    ngrp = batch // SUB
    win = G * SUB * FIELDS
    n = batch * FIELDS
    flat_tables = tables.reshape(FIELDS * VOCAB, EMB)
    offs = (jnp.arange(FIELDS, dtype=indices.dtype) * VOCAB)[None, :]
    pidx = (indices + offs).reshape(ngrp, SUB, FIELDS).transpose(0, 2, 1).reshape(n)

    mesh = plsc.VectorSubcoreMesh(core_axis_name="core", subcore_axis_name="subcore")

    @functools.partial(
        pl.kernel,
        out_type=jax.ShapeDtypeStruct((ngrp, FIELDS, SUB, EMB), tables.dtype),
        mesh=mesh,
        scratch_types=[pltpu.VMEM((win,), jnp.int32)],
    )
    def gather_kernel(x_hbm, i_hbm, o_hbm, idx_v):
        def body(grid_idx, o_vmem):
            (i,) = grid_idx
            pltpu.sync_copy(i_hbm.at[pl.ds(i * win, win)], idx_v)
            pltpu.sync_copy(x_hbm.at[idx_v], o_vmem.reshape(win, EMB))

        pltpu.emit_pipeline(
            body,
            grid=(ngrp // G,),
            out_specs=[
                pl.BlockSpec((G, FIELDS, SUB, EMB), index_map=lambda i: (i, 0, 0, 0))
            ],
            core_axis_name=("core", "subcore"),
            dimension_semantics=(pltpu.PARALLEL,),
            _explicit_indices=True,
        )(o_hbm)

    out4 = gather_kernel(flat_tables, pidx)
    return out4.transpose(0, 2, 1, 3).reshape(batch, FIELDS * EMB)
